# baseline (speedup 1.0000x reference)
import functools

import jax
import jax.numpy as jnp
from jax import lax
from jax.experimental import pallas as pl
from jax.experimental.pallas import tpu as pltpu
from jax.experimental.pallas import tpu_sc as plsc

N = 10000
D_IN = 128
H = 64
E = 320000

NC = 2
NS = 16
NW = NC * NS

LCHUNK = 128
CPW = 80
EP = NW * LCHUNK * CPW
C0PW = 160
C1PW = 0
DUMMY = N

NP = 10240
RPS = NP // NS
BLK = 512
NBLK = NP // BLK
DEGW = 16

_MESH = dict(core_axis_name="c", subcore_axis_name="s",
             num_cores=NC, num_subcores=NS)

_SC_PARAMS = pltpu.CompilerParams(use_tc_tiling_on_sc=False)


def _sc_degree(dst2d, ones_d, zeros_d):
    @functools.partial(
        pl.kernel,
        out_type=jax.ShapeDtypeStruct((NC, NP, DEGW), jnp.float32),
        mesh=plsc.VectorSubcoreMesh(**_MESH),
        compiler_params=_SC_PARAMS,
        scratch_types=[
            pltpu.VMEM((CPW, LCHUNK), jnp.int32),
            pltpu.VMEM((LCHUNK, DEGW), jnp.float32),
            pltpu.VMEM_SHARED((NP, DEGW), jnp.float32),
        ],
    )
    def k(dst_hbm, ones_hbm, zeros_hbm, out_hbm, dst_v, ones_v, acc):
        cid = lax.axis_index("c")
        sid = lax.axis_index("s")
        wid = sid * NC + cid
        r0 = sid * RPS
        pltpu.sync_copy(zeros_hbm.at[pl.ds(r0, RPS)], acc.at[pl.ds(r0, RPS)])
        pltpu.sync_copy(ones_hbm, ones_v)
        pltpu.sync_copy(dst_hbm.at[pl.ds(wid * CPW, CPW)], dst_v)
        plsc.subcore_barrier()

        @pl.loop(0, CPW)
        def _(j):
            pltpu.sync_copy(ones_v, acc.at[dst_v.at[j]], add=True)

        plsc.subcore_barrier()
        pltpu.sync_copy(acc.at[pl.ds(r0, RPS)],
                        out_hbm.at[cid].at[pl.ds(r0, RPS)])

    return k(dst2d, ones_d, zeros_d)


def _sc_scatter(table, src2d, dst2d, zeros_h):
    @functools.partial(
        pl.kernel,
        out_type=jax.ShapeDtypeStruct((NC, NP, H), jnp.float32),
        mesh=plsc.VectorSubcoreMesh(**_MESH),
        compiler_params=_SC_PARAMS,
        scratch_types=[
            pltpu.VMEM((C0PW, LCHUNK), jnp.int32),
            pltpu.VMEM((C0PW, LCHUNK), jnp.int32),
            [pltpu.VMEM((LCHUNK, H), jnp.float32) for _ in range(4)],
            pltpu.VMEM_SHARED((NP, H), jnp.float32),
            [pltpu.SemaphoreType.DMA for _ in range(4)],
        ],
    )
    def k(table_hbm, src_hbm, dst_hbm, zeros_hbm, out_hbm,
          src_v, dst_v, rows_bufs, acc, sems):
        cid = lax.axis_index("c")
        sid = lax.axis_index("s")
        r0 = sid * RPS
        cpw = jnp.where(cid == 0, C0PW, C1PW)
        pltpu.sync_copy(zeros_hbm.at[pl.ds(r0, RPS)], acc.at[pl.ds(r0, RPS)])

        @pl.when(cid == 0)
        def _():
            b0 = sid * C0PW
            pltpu.sync_copy(src_hbm.at[pl.ds(b0, C0PW)], src_v)
            pltpu.sync_copy(dst_hbm.at[pl.ds(b0, C0PW)], dst_v)

        plsc.subcore_barrier()

        @pl.when(cid == 0)
        def _():
            for b in range(4):
                pltpu.async_copy(table_hbm.at[src_v.at[b]],
                                 rows_bufs[b], sems[b])

            @pl.loop(0, cpw, step=4)
            def _(j):
                for b in range(4):
                    pltpu.make_async_copy(table_hbm.at[src_v.at[0]],
                                          rows_bufs[b], sems[b]).wait()
                    pltpu.sync_copy(rows_bufs[b], acc.at[dst_v.at[j + b]],
                                    add=True)

                    @pl.when(j + b + 4 < cpw)
                    def _():
                        pltpu.async_copy(table_hbm.at[src_v.at[j + b + 4]],
                                         rows_bufs[b], sems[b])

        plsc.subcore_barrier()
        pltpu.sync_copy(acc.at[pl.ds(r0, RPS)],
                        out_hbm.at[cid].at[pl.ds(r0, RPS)])

    return k(table, src2d, dst2d, zeros_h)


def _dinv_block(d_ref):
    s = jnp.sum(d_ref[0] + d_ref[1], axis=1, keepdims=True)
    return lax.rsqrt(s * (1.0 / DEGW) + 1.0)


_PREC = lax.Precision.HIGHEST


def _tc_prep(x_pad, W1, degp):
    def body(x_ref, w_ref, d_ref, g_ref):
        dinv = _dinv_block(d_ref)
        g_ref[...] = jnp.dot(x_ref[...], w_ref[...],
                             preferred_element_type=jnp.float32,
                             precision=_PREC) * dinv

    return pl.pallas_call(
        body,
        grid=(NBLK,),
        in_specs=[
            pl.BlockSpec((BLK, D_IN), lambda i: (i, 0)),
            pl.BlockSpec((D_IN, H), lambda i: (0, 0)),
            pl.BlockSpec((NC, BLK, DEGW), lambda i: (0, i, 0)),
        ],
        out_specs=pl.BlockSpec((BLK, H), lambda i: (i, 0)),
        out_shape=jax.ShapeDtypeStruct((NP, H), jnp.float32),
    )(x_pad, W1, degp)


def _pre_block(s_ref, g_ref, d_ref, b_ref):
    dinv = _dinv_block(d_ref)
    return dinv * (s_ref[0] + s_ref[1] + g_ref[...]) + b_ref[...]


def _tc_stats(Sp, g, degp, b2d):
    def body(s_ref, g_ref, d_ref, b_ref, o_ref, acc_ref):
        i = pl.program_id(0)

        @pl.when(i == 0)
        def _():
            acc_ref[0] = 0.0
            acc_ref[1] = 0.0

        pre = _pre_block(s_ref, g_ref, d_ref, b_ref)
        rows = lax.broadcasted_iota(jnp.int32, (BLK, H), 0) + i * BLK
        prem = jnp.where(rows < N, pre, 0.0)
        acc_ref[0] += jnp.sum(prem)
        acc_ref[1] += jnp.sum(prem * prem)

        @pl.when(i == NBLK - 1)
        def _():
            cnt = float(N * H)
            mean = acc_ref[0] / cnt
            var = acc_ref[1] / cnt - mean * mean
            o_ref[0] = mean
            o_ref[1] = lax.rsqrt(var + 1e-5)

    return pl.pallas_call(
        body,
        grid=(NBLK,),
        in_specs=[
            pl.BlockSpec((NC, BLK, H), lambda i: (0, i, 0)),
            pl.BlockSpec((BLK, H), lambda i: (i, 0)),
            pl.BlockSpec((NC, BLK, DEGW), lambda i: (0, i, 0)),
            pl.BlockSpec((1, H), lambda i: (0, 0)),
        ],
        out_specs=pl.BlockSpec(memory_space=pltpu.SMEM),
        out_shape=jax.ShapeDtypeStruct((2,), jnp.float32),
        scratch_shapes=[pltpu.SMEM((2,), jnp.float32)],
    )(Sp, g, degp, b2d)


def _tc_epilogue(Sp, g, degp, b2d, st, lnw2d, lnb2d, h_prev, W_next):
    residual = h_prev is not None

    def body(s_ref, g_ref, d_ref, b_ref, lnw_ref, lnb_ref, *rest):
        if residual:
            hp_ref, w_ref, st_ref, h_ref, gn_ref = rest
        else:
            w_ref, st_ref, h_ref, gn_ref = rest
        dinv = _dinv_block(d_ref)
        pre = dinv * (s_ref[0] + s_ref[1] + g_ref[...]) + b_ref[...]
        y = (pre - st_ref[0]) * st_ref[1] * lnw_ref[...] + lnb_ref[...]
        h = jnp.maximum(y, 0.0)
        if residual:
            h = h + hp_ref[...]
        h_ref[...] = h
        gn_ref[...] = jnp.dot(h, w_ref[...],
                              preferred_element_type=jnp.float32,
                              precision=_PREC) * dinv

    in_specs = [
        pl.BlockSpec((NC, BLK, H), lambda i: (0, i, 0)),
        pl.BlockSpec((BLK, H), lambda i: (i, 0)),
        pl.BlockSpec((NC, BLK, DEGW), lambda i: (0, i, 0)),
        pl.BlockSpec((1, H), lambda i: (0, 0)),
        pl.BlockSpec((1, H), lambda i: (0, 0)),
        pl.BlockSpec((1, H), lambda i: (0, 0)),
    ]
    args = [Sp, g, degp, b2d, lnw2d, lnb2d]
    if residual:
        in_specs.append(pl.BlockSpec((BLK, H), lambda i: (i, 0)))
        args.append(h_prev)
    in_specs.append(pl.BlockSpec((H, H), lambda i: (0, 0)))
    args.append(W_next)
    in_specs.append(pl.BlockSpec(memory_space=pltpu.SMEM))
    args.append(st)

    return pl.pallas_call(
        body,
        grid=(NBLK,),
        in_specs=in_specs,
        out_specs=[
            pl.BlockSpec((BLK, H), lambda i: (i, 0)),
            pl.BlockSpec((BLK, H), lambda i: (i, 0)),
        ],
        out_shape=[
            jax.ShapeDtypeStruct((NP, H), jnp.float32),
            jax.ShapeDtypeStruct((NP, H), jnp.float32),
        ],
    )(*args)


def _tc_epilogue_head(Sp, g, degp, b2d, st, lnw2d, lnb2d, h_prev,
                      Wp1, bp1_2d, Wp2_row, bp2):
    def body(s_ref, g_ref, d_ref, b_ref, lnw_ref, lnb_ref, hp_ref,
             wp1_ref, bp1_ref, wp2_ref, st_ref, bp2_ref,
             emb_ref, pk_ref):
        pre = _pre_block(s_ref, g_ref, d_ref, b_ref)
        y = (pre - st_ref[0]) * st_ref[1] * lnw_ref[...] + lnb_ref[...]
        h = jnp.maximum(y, 0.0) + hp_ref[...]
        emb_ref[...] = h
        t = jnp.maximum(
            jnp.dot(h, wp1_ref[...], preferred_element_type=jnp.float32,
                    precision=_PREC) + bp1_ref[...], 0.0)
        pk_ref[...] = jnp.sum(t * wp2_ref[...], axis=1) + bp2_ref[0]

    return pl.pallas_call(
        body,
        grid=(NBLK,),
        in_specs=[
            pl.BlockSpec((NC, BLK, H), lambda i: (0, i, 0)),
            pl.BlockSpec((BLK, H), lambda i: (i, 0)),
            pl.BlockSpec((NC, BLK, DEGW), lambda i: (0, i, 0)),
            pl.BlockSpec((1, H), lambda i: (0, 0)),
            pl.BlockSpec((1, H), lambda i: (0, 0)),
            pl.BlockSpec((1, H), lambda i: (0, 0)),
            pl.BlockSpec((BLK, H), lambda i: (i, 0)),
            pl.BlockSpec((H, H // 2), lambda i: (0, 0)),
            pl.BlockSpec((1, H // 2), lambda i: (0, 0)),
            pl.BlockSpec((1, H // 2), lambda i: (0, 0)),
            pl.BlockSpec(memory_space=pltpu.SMEM),
            pl.BlockSpec(memory_space=pltpu.SMEM),
        ],
        out_specs=[
            pl.BlockSpec((BLK, H), lambda i: (i, 0)),
            pl.BlockSpec((BLK,), lambda i: (i,)),
        ],
        out_shape=[
            jax.ShapeDtypeStruct((NP, H), jnp.float32),
            jax.ShapeDtypeStruct((NP,), jnp.float32),
        ],
    )(Sp, g, degp, b2d, lnw2d, lnb2d, h_prev, Wp1, bp1_2d, Wp2_row, st, bp2)


def kernel(x, edge_index, W1, b1, ln_w1, ln_b1, W2, b2, ln_w2, ln_b2,
           W3, b3, ln_w3, ln_b3, Wp1, bp1, Wp2, bp2):
    src = edge_index[0]
    dst = edge_index[1]
    pad = jnp.full((EP - E,), DUMMY, dtype=src.dtype)
    src2d = jnp.concatenate([src, pad]).reshape(NW * CPW, LCHUNK)
    dst2d = jnp.concatenate([dst, pad]).reshape(NW * CPW, LCHUNK)

    x_pad = jnp.pad(x, ((0, NP - N), (0, 0)))
    zeros_h = jnp.zeros((NP, H), jnp.float32)
    zeros_d = jnp.zeros((NP, DEGW), jnp.float32)
    ones_d = jnp.ones((LCHUNK, DEGW), jnp.float32)

    b1r = b1.reshape(1, H)
    b2r = b2.reshape(1, H)
    b3r = b3.reshape(1, H)
    lnw1r = ln_w1.reshape(1, H)
    lnb1r = ln_b1.reshape(1, H)
    lnw2r = ln_w2.reshape(1, H)
    lnb2r = ln_b2.reshape(1, H)
    lnw3r = ln_w3.reshape(1, H)
    lnb3r = ln_b3.reshape(1, H)
    bp1r = bp1.reshape(1, H // 2)
    wp2r = Wp2.reshape(1, H // 2)

    degp = _sc_degree(dst2d, ones_d, zeros_d)

    g1 = _tc_prep(x_pad, W1, degp)
    S1 = _sc_scatter(g1, src2d, dst2d, zeros_h)
    st1 = _tc_stats(S1, g1, degp, b1r)
    h1, g2 = _tc_epilogue(S1, g1, degp, b1r, st1, lnw1r, lnb1r, None, W2)

    S2 = _sc_scatter(g2, src2d, dst2d, zeros_h)
    st2 = _tc_stats(S2, g2, degp, b2r)
    h2, g3 = _tc_epilogue(S2, g2, degp, b2r, st2, lnw2r, lnb2r, h1, W3)

    S3 = _sc_scatter(g3, src2d, dst2d, zeros_h)
    st3 = _tc_stats(S3, g3, degp, b3r)
    emb, pk = _tc_epilogue_head(S3, g3, degp, b3r, st3, lnw3r, lnb3r, h2,
                                Wp1, bp1r, wp2r, bp2)

    return (emb[:N], pk[:N, None])


# speedup vs baseline: 1.3205x; 1.3205x over previous
import functools

import jax
import jax.numpy as jnp
from jax import lax
from jax.experimental import pallas as pl
from jax.experimental.pallas import tpu as pltpu
from jax.experimental.pallas import tpu_sc as plsc

N = 10000
D_IN = 128
H = 64
E = 320000

NC = 2
NS = 16
NW = NC * NS

LCHUNK = 128
CPW = 80
EP = NW * LCHUNK * CPW
C0PW = 152
C1PW = 8
DUMMY = N

NP = 10240
RPS = NP // NS
BLK = 512
NBLK = NP // BLK
DEGW = 16

_MESH = dict(core_axis_name="c", subcore_axis_name="s",
             num_cores=NC, num_subcores=NS)

_SC_PARAMS = pltpu.CompilerParams(use_tc_tiling_on_sc=False)


def _sc_degree(dst2d, ones_d, zeros_d):
    @functools.partial(
        pl.kernel,
        out_type=jax.ShapeDtypeStruct((NC, NP, DEGW), jnp.float32),
        mesh=plsc.VectorSubcoreMesh(**_MESH),
        compiler_params=_SC_PARAMS,
        scratch_types=[
            pltpu.VMEM((CPW, LCHUNK), jnp.int32),
            pltpu.VMEM((LCHUNK, DEGW), jnp.float32),
            pltpu.VMEM_SHARED((NP, DEGW), jnp.float32),
        ],
    )
    def k(dst_hbm, ones_hbm, zeros_hbm, out_hbm, dst_v, ones_v, acc):
        cid = lax.axis_index("c")
        sid = lax.axis_index("s")
        wid = sid * NC + cid
        r0 = sid * RPS
        pltpu.sync_copy(zeros_hbm.at[pl.ds(r0, RPS)], acc.at[pl.ds(r0, RPS)])
        pltpu.sync_copy(ones_hbm, ones_v)
        pltpu.sync_copy(dst_hbm.at[pl.ds(wid * CPW, CPW)], dst_v)
        plsc.subcore_barrier()

        @pl.loop(0, CPW)
        def _(j):
            pltpu.sync_copy(ones_v, acc.at[dst_v.at[j]], add=True)

        plsc.subcore_barrier()
        pltpu.sync_copy(acc.at[pl.ds(r0, RPS)],
                        out_hbm.at[cid].at[pl.ds(r0, RPS)])

    return k(dst2d, ones_d, zeros_d)


def _sc_scatter(table, src2d, dst2d, zeros_h):
    @functools.partial(
        pl.kernel,
        out_type=jax.ShapeDtypeStruct((NC, NP, H), jnp.float32),
        mesh=plsc.VectorSubcoreMesh(**_MESH),
        compiler_params=_SC_PARAMS,
        scratch_types=[
            pltpu.VMEM((C0PW, LCHUNK), jnp.int32),
            pltpu.VMEM((C0PW, LCHUNK), jnp.int32),
            [pltpu.VMEM((LCHUNK, H), jnp.float32) for _ in range(4)],
            pltpu.VMEM_SHARED((NP, H), jnp.float32),
            [pltpu.SemaphoreType.DMA for _ in range(4)],
        ],
    )
    def k(table_hbm, src_hbm, dst_hbm, zeros_hbm, out_hbm,
          src_v, dst_v, rows_bufs, acc, sems):
        cid = lax.axis_index("c")
        sid = lax.axis_index("s")
        r0 = sid * RPS
        cpw = jnp.where(cid == 0, C0PW, C1PW)
        pltpu.sync_copy(zeros_hbm.at[pl.ds(r0, RPS)], acc.at[pl.ds(r0, RPS)])

        @pl.when(cid == 0)
        def _():
            b0 = sid * C0PW
            pltpu.sync_copy(src_hbm.at[pl.ds(b0, C0PW)], src_v)
            pltpu.sync_copy(dst_hbm.at[pl.ds(b0, C0PW)], dst_v)

        @pl.when(cid == 1)
        def _():
            b1 = NS * C0PW + sid * C1PW
            pltpu.sync_copy(src_hbm.at[pl.ds(b1, C1PW)],
                            src_v.at[pl.ds(0, C1PW)])
            pltpu.sync_copy(dst_hbm.at[pl.ds(b1, C1PW)],
                            dst_v.at[pl.ds(0, C1PW)])

        plsc.subcore_barrier()

        for b in range(4):
            pltpu.async_copy(table_hbm.at[src_v.at[b]], rows_bufs[b], sems[b])

        @pl.loop(0, cpw, step=4)
        def _(j):
            for b in range(4):
                pltpu.make_async_copy(table_hbm.at[src_v.at[0]],
                                      rows_bufs[b], sems[b]).wait()
                pltpu.sync_copy(rows_bufs[b], acc.at[dst_v.at[j + b]],
                                add=True)

                @pl.when(j + b + 4 < cpw)
                def _():
                    pltpu.async_copy(table_hbm.at[src_v.at[j + b + 4]],
                                     rows_bufs[b], sems[b])

        plsc.subcore_barrier()
        pltpu.sync_copy(acc.at[pl.ds(r0, RPS)],
                        out_hbm.at[cid].at[pl.ds(r0, RPS)])

    return k(table, src2d, dst2d, zeros_h)


def _dinv_block(d_ref):
    s = jnp.sum(d_ref[0] + d_ref[1], axis=1, keepdims=True)
    return lax.rsqrt(s * (1.0 / DEGW) + 1.0)


_PREC = lax.Precision.HIGHEST


def _tc_prep(x_pad, W1, degp):
    def body(x_ref, w_ref, d_ref, g_ref):
        dinv = _dinv_block(d_ref)
        g_ref[...] = jnp.dot(x_ref[...], w_ref[...],
                             preferred_element_type=jnp.float32,
                             precision=_PREC) * dinv

    return pl.pallas_call(
        body,
        grid=(NBLK,),
        in_specs=[
            pl.BlockSpec((BLK, D_IN), lambda i: (i, 0)),
            pl.BlockSpec((D_IN, H), lambda i: (0, 0)),
            pl.BlockSpec((NC, BLK, DEGW), lambda i: (0, i, 0)),
        ],
        out_specs=pl.BlockSpec((BLK, H), lambda i: (i, 0)),
        out_shape=jax.ShapeDtypeStruct((NP, H), jnp.float32),
    )(x_pad, W1, degp)


def _pre_block(s_ref, g_ref, d_ref, b_ref):
    dinv = _dinv_block(d_ref)
    return dinv * (s_ref[0] + s_ref[1] + g_ref[...]) + b_ref[...]


def _tc_stats(Sp, g, degp, b2d):
    def body(s_ref, g_ref, d_ref, b_ref, o_ref, acc_ref):
        i = pl.program_id(0)

        @pl.when(i == 0)
        def _():
            acc_ref[0] = 0.0
            acc_ref[1] = 0.0

        pre = _pre_block(s_ref, g_ref, d_ref, b_ref)
        rows = lax.broadcasted_iota(jnp.int32, (BLK, H), 0) + i * BLK
        prem = jnp.where(rows < N, pre, 0.0)
        acc_ref[0] += jnp.sum(prem)
        acc_ref[1] += jnp.sum(prem * prem)

        @pl.when(i == NBLK - 1)
        def _():
            cnt = float(N * H)
            mean = acc_ref[0] / cnt
            var = acc_ref[1] / cnt - mean * mean
            o_ref[0] = mean
            o_ref[1] = lax.rsqrt(var + 1e-5)

    return pl.pallas_call(
        body,
        grid=(NBLK,),
        in_specs=[
            pl.BlockSpec((NC, BLK, H), lambda i: (0, i, 0)),
            pl.BlockSpec((BLK, H), lambda i: (i, 0)),
            pl.BlockSpec((NC, BLK, DEGW), lambda i: (0, i, 0)),
            pl.BlockSpec((1, H), lambda i: (0, 0)),
        ],
        out_specs=pl.BlockSpec(memory_space=pltpu.SMEM),
        out_shape=jax.ShapeDtypeStruct((2,), jnp.float32),
        scratch_shapes=[pltpu.SMEM((2,), jnp.float32)],
    )(Sp, g, degp, b2d)


def _tc_epilogue(Sp, g, degp, b2d, st, lnw2d, lnb2d, h_prev, W_next):
    residual = h_prev is not None

    def body(s_ref, g_ref, d_ref, b_ref, lnw_ref, lnb_ref, *rest):
        if residual:
            hp_ref, w_ref, st_ref, h_ref, gn_ref = rest
        else:
            w_ref, st_ref, h_ref, gn_ref = rest
        dinv = _dinv_block(d_ref)
        pre = dinv * (s_ref[0] + s_ref[1] + g_ref[...]) + b_ref[...]
        y = (pre - st_ref[0]) * st_ref[1] * lnw_ref[...] + lnb_ref[...]
        h = jnp.maximum(y, 0.0)
        if residual:
            h = h + hp_ref[...]
        h_ref[...] = h
        gn_ref[...] = jnp.dot(h, w_ref[...],
                              preferred_element_type=jnp.float32,
                              precision=_PREC) * dinv

    in_specs = [
        pl.BlockSpec((NC, BLK, H), lambda i: (0, i, 0)),
        pl.BlockSpec((BLK, H), lambda i: (i, 0)),
        pl.BlockSpec((NC, BLK, DEGW), lambda i: (0, i, 0)),
        pl.BlockSpec((1, H), lambda i: (0, 0)),
        pl.BlockSpec((1, H), lambda i: (0, 0)),
        pl.BlockSpec((1, H), lambda i: (0, 0)),
    ]
    args = [Sp, g, degp, b2d, lnw2d, lnb2d]
    if residual:
        in_specs.append(pl.BlockSpec((BLK, H), lambda i: (i, 0)))
        args.append(h_prev)
    in_specs.append(pl.BlockSpec((H, H), lambda i: (0, 0)))
    args.append(W_next)
    in_specs.append(pl.BlockSpec(memory_space=pltpu.SMEM))
    args.append(st)

    return pl.pallas_call(
        body,
        grid=(NBLK,),
        in_specs=in_specs,
        out_specs=[
            pl.BlockSpec((BLK, H), lambda i: (i, 0)),
            pl.BlockSpec((BLK, H), lambda i: (i, 0)),
        ],
        out_shape=[
            jax.ShapeDtypeStruct((NP, H), jnp.float32),
            jax.ShapeDtypeStruct((NP, H), jnp.float32),
        ],
    )(*args)


def _tc_epilogue_head(Sp, g, degp, b2d, st, lnw2d, lnb2d, h_prev,
                      Wp1, bp1_2d, Wp2_row, bp2):
    def body(s_ref, g_ref, d_ref, b_ref, lnw_ref, lnb_ref, hp_ref,
             wp1_ref, bp1_ref, wp2_ref, st_ref, bp2_ref,
             emb_ref, pk_ref):
        pre = _pre_block(s_ref, g_ref, d_ref, b_ref)
        y = (pre - st_ref[0]) * st_ref[1] * lnw_ref[...] + lnb_ref[...]
        h = jnp.maximum(y, 0.0) + hp_ref[...]
        emb_ref[...] = h
        t = jnp.maximum(
            jnp.dot(h, wp1_ref[...], preferred_element_type=jnp.float32,
                    precision=_PREC) + bp1_ref[...], 0.0)
        pk_ref[...] = jnp.sum(t * wp2_ref[...], axis=1) + bp2_ref[0]

    return pl.pallas_call(
        body,
        grid=(NBLK,),
        in_specs=[
            pl.BlockSpec((NC, BLK, H), lambda i: (0, i, 0)),
            pl.BlockSpec((BLK, H), lambda i: (i, 0)),
            pl.BlockSpec((NC, BLK, DEGW), lambda i: (0, i, 0)),
            pl.BlockSpec((1, H), lambda i: (0, 0)),
            pl.BlockSpec((1, H), lambda i: (0, 0)),
            pl.BlockSpec((1, H), lambda i: (0, 0)),
            pl.BlockSpec((BLK, H), lambda i: (i, 0)),
            pl.BlockSpec((H, H // 2), lambda i: (0, 0)),
            pl.BlockSpec((1, H // 2), lambda i: (0, 0)),
            pl.BlockSpec((1, H // 2), lambda i: (0, 0)),
            pl.BlockSpec(memory_space=pltpu.SMEM),
            pl.BlockSpec(memory_space=pltpu.SMEM),
        ],
        out_specs=[
            pl.BlockSpec((BLK, H), lambda i: (i, 0)),
            pl.BlockSpec((BLK,), lambda i: (i,)),
        ],
        out_shape=[
            jax.ShapeDtypeStruct((NP, H), jnp.float32),
            jax.ShapeDtypeStruct((NP,), jnp.float32),
        ],
    )(Sp, g, degp, b2d, lnw2d, lnb2d, h_prev, Wp1, bp1_2d, Wp2_row, st, bp2)


def kernel(x, edge_index, W1, b1, ln_w1, ln_b1, W2, b2, ln_w2, ln_b2,
           W3, b3, ln_w3, ln_b3, Wp1, bp1, Wp2, bp2):
    src = edge_index[0]
    dst = edge_index[1]
    pad = jnp.full((EP - E,), DUMMY, dtype=src.dtype)
    src2d = jnp.concatenate([src, pad]).reshape(NW * CPW, LCHUNK)
    dst2d = jnp.concatenate([dst, pad]).reshape(NW * CPW, LCHUNK)

    x_pad = jnp.pad(x, ((0, NP - N), (0, 0)))
    zeros_h = jnp.zeros((NP, H), jnp.float32)
    zeros_d = jnp.zeros((NP, DEGW), jnp.float32)
    ones_d = jnp.ones((LCHUNK, DEGW), jnp.float32)

    b1r = b1.reshape(1, H)
    b2r = b2.reshape(1, H)
    b3r = b3.reshape(1, H)
    lnw1r = ln_w1.reshape(1, H)
    lnb1r = ln_b1.reshape(1, H)
    lnw2r = ln_w2.reshape(1, H)
    lnb2r = ln_b2.reshape(1, H)
    lnw3r = ln_w3.reshape(1, H)
    lnb3r = ln_b3.reshape(1, H)
    bp1r = bp1.reshape(1, H // 2)
    wp2r = Wp2.reshape(1, H // 2)

    degp = _sc_degree(dst2d, ones_d, zeros_d)

    g1 = _tc_prep(x_pad, W1, degp)
    S1 = _sc_scatter(g1, src2d, dst2d, zeros_h)
    st1 = _tc_stats(S1, g1, degp, b1r)
    h1, g2 = _tc_epilogue(S1, g1, degp, b1r, st1, lnw1r, lnb1r, None, W2)

    S2 = _sc_scatter(g2, src2d, dst2d, zeros_h)
    st2 = _tc_stats(S2, g2, degp, b2r)
    h2, g3 = _tc_epilogue(S2, g2, degp, b2r, st2, lnw2r, lnb2r, h1, W3)

    S3 = _sc_scatter(g3, src2d, dst2d, zeros_h)
    st3 = _tc_stats(S3, g3, degp, b3r)
    emb, pk = _tc_epilogue_head(S3, g3, degp, b3r, st3, lnw3r, lnb3r, h2,
                                Wp1, bp1r, wp2r, bp2)

    return (emb[:N], pk[:N, None])


# speedup vs baseline: 1.3321x; 1.0088x over previous
import functools

import jax
import jax.numpy as jnp
from jax import lax
from jax.experimental import pallas as pl
from jax.experimental.pallas import tpu as pltpu
from jax.experimental.pallas import tpu_sc as plsc

N = 10000
D_IN = 128
H = 64
E = 320000

NC = 2
NS = 16
NW = NC * NS

LCHUNK = 128
CPW = 80
EP = NW * LCHUNK * CPW
C0PW = 152
C1PW = 8
DUMMY = N

NP = 10240
RPS = NP // NS
BLK = 512
NBLK = NP // BLK
DEGW = 16

_MESH = dict(core_axis_name="c", subcore_axis_name="s",
             num_cores=NC, num_subcores=NS)

_SC_PARAMS = pltpu.CompilerParams(use_tc_tiling_on_sc=False)


def _sc_degree(dst2d, ones_d, zeros_d):
    @functools.partial(
        pl.kernel,
        out_type=jax.ShapeDtypeStruct((NC, NP, DEGW), jnp.float32),
        mesh=plsc.VectorSubcoreMesh(**_MESH),
        compiler_params=_SC_PARAMS,
        scratch_types=[
            pltpu.VMEM((CPW, LCHUNK), jnp.int32),
            pltpu.VMEM((LCHUNK, DEGW), jnp.float32),
            pltpu.VMEM_SHARED((NP, DEGW), jnp.float32),
        ],
    )
    def k(dst_hbm, ones_hbm, zeros_hbm, out_hbm, dst_v, ones_v, acc):
        cid = lax.axis_index("c")
        sid = lax.axis_index("s")
        wid = sid * NC + cid
        r0 = sid * RPS
        pltpu.sync_copy(zeros_hbm.at[pl.ds(r0, RPS)], acc.at[pl.ds(r0, RPS)])
        pltpu.sync_copy(ones_hbm, ones_v)
        pltpu.sync_copy(dst_hbm.at[pl.ds(wid * CPW, CPW)], dst_v)
        plsc.subcore_barrier()

        @pl.loop(0, CPW)
        def _(j):
            pltpu.sync_copy(ones_v, acc.at[dst_v.at[j]], add=True)

        plsc.subcore_barrier()
        pltpu.sync_copy(acc.at[pl.ds(r0, RPS)],
                        out_hbm.at[cid].at[pl.ds(r0, RPS)])

    return k(dst2d, ones_d, zeros_d)


def _sc_scatter(table, src2d, dst2d, zeros_h):
    @functools.partial(
        pl.kernel,
        out_type=jax.ShapeDtypeStruct((NC, NP, H), jnp.float32),
        mesh=plsc.VectorSubcoreMesh(**_MESH),
        compiler_params=_SC_PARAMS,
        scratch_types=[
            pltpu.VMEM((C0PW, LCHUNK), jnp.int32),
            pltpu.VMEM((C0PW, LCHUNK), jnp.int32),
            [pltpu.VMEM((LCHUNK, H), jnp.float32) for _ in range(4)],
            pltpu.VMEM_SHARED((NP, H), jnp.float32),
            [pltpu.SemaphoreType.DMA for _ in range(4)],
        ],
    )
    def k(table_hbm, src_hbm, dst_hbm, zeros_hbm, out_hbm,
          src_v, dst_v, rows_bufs, acc, sems):
        cid = lax.axis_index("c")
        sid = lax.axis_index("s")
        r0 = sid * RPS
        cpw = jnp.where(cid == 0, C0PW, C1PW)
        pltpu.sync_copy(zeros_hbm.at[pl.ds(r0, RPS)], acc.at[pl.ds(r0, RPS)])

        @pl.when(cid == 0)
        def _():
            b0 = sid * C0PW
            pltpu.sync_copy(src_hbm.at[pl.ds(b0, C0PW)], src_v)
            pltpu.sync_copy(dst_hbm.at[pl.ds(b0, C0PW)], dst_v)

        @pl.when(cid == 1)
        def _():
            b1 = NS * C0PW + sid * C1PW
            pltpu.sync_copy(src_hbm.at[pl.ds(b1, C1PW)],
                            src_v.at[pl.ds(0, C1PW)])
            pltpu.sync_copy(dst_hbm.at[pl.ds(b1, C1PW)],
                            dst_v.at[pl.ds(0, C1PW)])

        plsc.subcore_barrier()

        for b in range(4):
            pltpu.async_copy(table_hbm.at[src_v.at[b]], rows_bufs[b], sems[b])

        @pl.loop(0, cpw, step=4)
        def _(j):
            for b in range(4):
                pltpu.make_async_copy(table_hbm.at[src_v.at[0]],
                                      rows_bufs[b], sems[b]).wait()
                pltpu.sync_copy(rows_bufs[b], acc.at[dst_v.at[j + b]],
                                add=True)

                @pl.when(j + b + 4 < cpw)
                def _():
                    pltpu.async_copy(table_hbm.at[src_v.at[j + b + 4]],
                                     rows_bufs[b], sems[b])

        plsc.subcore_barrier()
        pltpu.sync_copy(acc.at[pl.ds(r0, RPS)],
                        out_hbm.at[cid].at[pl.ds(r0, RPS)])

    return k(table, src2d, dst2d, zeros_h)


def _dinv_block(d_ref):
    s = jnp.sum(d_ref[0] + d_ref[1], axis=1, keepdims=True)
    return lax.rsqrt(s * (1.0 / DEGW) + 1.0)


_PREC = None


def _tc_prep(x_pad, W1, degp):
    def body(x_ref, w_ref, d_ref, g_ref):
        dinv = _dinv_block(d_ref)
        g_ref[...] = jnp.dot(x_ref[...], w_ref[...],
                             preferred_element_type=jnp.float32,
                             precision=_PREC) * dinv

    return pl.pallas_call(
        body,
        grid=(NBLK,),
        in_specs=[
            pl.BlockSpec((BLK, D_IN), lambda i: (i, 0)),
            pl.BlockSpec((D_IN, H), lambda i: (0, 0)),
            pl.BlockSpec((NC, BLK, DEGW), lambda i: (0, i, 0)),
        ],
        out_specs=pl.BlockSpec((BLK, H), lambda i: (i, 0)),
        out_shape=jax.ShapeDtypeStruct((NP, H), jnp.float32),
    )(x_pad, W1, degp)


def _pre_block(s_ref, g_ref, d_ref, b_ref):
    dinv = _dinv_block(d_ref)
    return dinv * (s_ref[0] + s_ref[1] + g_ref[...]) + b_ref[...]


def _tc_stats(Sp, g, degp, b2d):
    def body(s_ref, g_ref, d_ref, b_ref, o_ref, acc_ref):
        i = pl.program_id(0)

        @pl.when(i == 0)
        def _():
            acc_ref[0] = 0.0
            acc_ref[1] = 0.0

        pre = _pre_block(s_ref, g_ref, d_ref, b_ref)
        rows = lax.broadcasted_iota(jnp.int32, (BLK, H), 0) + i * BLK
        prem = jnp.where(rows < N, pre, 0.0)
        acc_ref[0] += jnp.sum(prem)
        acc_ref[1] += jnp.sum(prem * prem)

        @pl.when(i == NBLK - 1)
        def _():
            cnt = float(N * H)
            mean = acc_ref[0] / cnt
            var = acc_ref[1] / cnt - mean * mean
            o_ref[0] = mean
            o_ref[1] = lax.rsqrt(var + 1e-5)

    return pl.pallas_call(
        body,
        grid=(NBLK,),
        in_specs=[
            pl.BlockSpec((NC, BLK, H), lambda i: (0, i, 0)),
            pl.BlockSpec((BLK, H), lambda i: (i, 0)),
            pl.BlockSpec((NC, BLK, DEGW), lambda i: (0, i, 0)),
            pl.BlockSpec((1, H), lambda i: (0, 0)),
        ],
        out_specs=pl.BlockSpec(memory_space=pltpu.SMEM),
        out_shape=jax.ShapeDtypeStruct((2,), jnp.float32),
        scratch_shapes=[pltpu.SMEM((2,), jnp.float32)],
    )(Sp, g, degp, b2d)


def _tc_epilogue(Sp, g, degp, b2d, st, lnw2d, lnb2d, h_prev, W_next):
    residual = h_prev is not None

    def body(s_ref, g_ref, d_ref, b_ref, lnw_ref, lnb_ref, *rest):
        if residual:
            hp_ref, w_ref, st_ref, h_ref, gn_ref = rest
        else:
            w_ref, st_ref, h_ref, gn_ref = rest
        dinv = _dinv_block(d_ref)
        pre = dinv * (s_ref[0] + s_ref[1] + g_ref[...]) + b_ref[...]
        y = (pre - st_ref[0]) * st_ref[1] * lnw_ref[...] + lnb_ref[...]
        h = jnp.maximum(y, 0.0)
        if residual:
            h = h + hp_ref[...]
        h_ref[...] = h
        gn_ref[...] = jnp.dot(h, w_ref[...],
                              preferred_element_type=jnp.float32,
                              precision=_PREC) * dinv

    in_specs = [
        pl.BlockSpec((NC, BLK, H), lambda i: (0, i, 0)),
        pl.BlockSpec((BLK, H), lambda i: (i, 0)),
        pl.BlockSpec((NC, BLK, DEGW), lambda i: (0, i, 0)),
        pl.BlockSpec((1, H), lambda i: (0, 0)),
        pl.BlockSpec((1, H), lambda i: (0, 0)),
        pl.BlockSpec((1, H), lambda i: (0, 0)),
    ]
    args = [Sp, g, degp, b2d, lnw2d, lnb2d]
    if residual:
        in_specs.append(pl.BlockSpec((BLK, H), lambda i: (i, 0)))
        args.append(h_prev)
    in_specs.append(pl.BlockSpec((H, H), lambda i: (0, 0)))
    args.append(W_next)
    in_specs.append(pl.BlockSpec(memory_space=pltpu.SMEM))
    args.append(st)

    return pl.pallas_call(
        body,
        grid=(NBLK,),
        in_specs=in_specs,
        out_specs=[
            pl.BlockSpec((BLK, H), lambda i: (i, 0)),
            pl.BlockSpec((BLK, H), lambda i: (i, 0)),
        ],
        out_shape=[
            jax.ShapeDtypeStruct((NP, H), jnp.float32),
            jax.ShapeDtypeStruct((NP, H), jnp.float32),
        ],
    )(*args)


def _tc_epilogue_head(Sp, g, degp, b2d, st, lnw2d, lnb2d, h_prev,
                      Wp1, bp1_2d, Wp2_row, bp2):
    def body(s_ref, g_ref, d_ref, b_ref, lnw_ref, lnb_ref, hp_ref,
             wp1_ref, bp1_ref, wp2_ref, st_ref, bp2_ref,
             emb_ref, pk_ref):
        pre = _pre_block(s_ref, g_ref, d_ref, b_ref)
        y = (pre - st_ref[0]) * st_ref[1] * lnw_ref[...] + lnb_ref[...]
        h = jnp.maximum(y, 0.0) + hp_ref[...]
        emb_ref[...] = h
        t = jnp.maximum(
            jnp.dot(h, wp1_ref[...], preferred_element_type=jnp.float32,
                    precision=_PREC) + bp1_ref[...], 0.0)
        pk_ref[...] = jnp.sum(t * wp2_ref[...], axis=1) + bp2_ref[0]

    return pl.pallas_call(
        body,
        grid=(NBLK,),
        in_specs=[
            pl.BlockSpec((NC, BLK, H), lambda i: (0, i, 0)),
            pl.BlockSpec((BLK, H), lambda i: (i, 0)),
            pl.BlockSpec((NC, BLK, DEGW), lambda i: (0, i, 0)),
            pl.BlockSpec((1, H), lambda i: (0, 0)),
            pl.BlockSpec((1, H), lambda i: (0, 0)),
            pl.BlockSpec((1, H), lambda i: (0, 0)),
            pl.BlockSpec((BLK, H), lambda i: (i, 0)),
            pl.BlockSpec((H, H // 2), lambda i: (0, 0)),
            pl.BlockSpec((1, H // 2), lambda i: (0, 0)),
            pl.BlockSpec((1, H // 2), lambda i: (0, 0)),
            pl.BlockSpec(memory_space=pltpu.SMEM),
            pl.BlockSpec(memory_space=pltpu.SMEM),
        ],
        out_specs=[
            pl.BlockSpec((BLK, H), lambda i: (i, 0)),
            pl.BlockSpec((BLK,), lambda i: (i,)),
        ],
        out_shape=[
            jax.ShapeDtypeStruct((NP, H), jnp.float32),
            jax.ShapeDtypeStruct((NP,), jnp.float32),
        ],
    )(Sp, g, degp, b2d, lnw2d, lnb2d, h_prev, Wp1, bp1_2d, Wp2_row, st, bp2)


def kernel(x, edge_index, W1, b1, ln_w1, ln_b1, W2, b2, ln_w2, ln_b2,
           W3, b3, ln_w3, ln_b3, Wp1, bp1, Wp2, bp2):
    src = edge_index[0]
    dst = edge_index[1]
    pad = jnp.full((EP - E,), DUMMY, dtype=src.dtype)
    src2d = jnp.concatenate([src, pad]).reshape(NW * CPW, LCHUNK)
    dst2d = jnp.concatenate([dst, pad]).reshape(NW * CPW, LCHUNK)

    x_pad = jnp.pad(x, ((0, NP - N), (0, 0)))
    zeros_h = jnp.zeros((NP, H), jnp.float32)
    zeros_d = jnp.zeros((NP, DEGW), jnp.float32)
    ones_d = jnp.ones((LCHUNK, DEGW), jnp.float32)

    b1r = b1.reshape(1, H)
    b2r = b2.reshape(1, H)
    b3r = b3.reshape(1, H)
    lnw1r = ln_w1.reshape(1, H)
    lnb1r = ln_b1.reshape(1, H)
    lnw2r = ln_w2.reshape(1, H)
    lnb2r = ln_b2.reshape(1, H)
    lnw3r = ln_w3.reshape(1, H)
    lnb3r = ln_b3.reshape(1, H)
    bp1r = bp1.reshape(1, H // 2)
    wp2r = Wp2.reshape(1, H // 2)

    degp = _sc_degree(dst2d, ones_d, zeros_d)

    g1 = _tc_prep(x_pad, W1, degp)
    S1 = _sc_scatter(g1, src2d, dst2d, zeros_h)
    st1 = _tc_stats(S1, g1, degp, b1r)
    h1, g2 = _tc_epilogue(S1, g1, degp, b1r, st1, lnw1r, lnb1r, None, W2)

    S2 = _sc_scatter(g2, src2d, dst2d, zeros_h)
    st2 = _tc_stats(S2, g2, degp, b2r)
    h2, g3 = _tc_epilogue(S2, g2, degp, b2r, st2, lnw2r, lnb2r, h1, W3)

    S3 = _sc_scatter(g3, src2d, dst2d, zeros_h)
    st3 = _tc_stats(S3, g3, degp, b3r)
    emb, pk = _tc_epilogue_head(S3, g3, degp, b3r, st3, lnw3r, lnb3r, h2,
                                Wp1, bp1r, wp2r, bp2)

    return (emb[:N], pk[:N, None])


# speedup vs baseline: 2.0422x; 1.5331x over previous
import functools

import jax
import jax.numpy as jnp
from jax import lax
from jax.experimental import pallas as pl
from jax.experimental.pallas import tpu as pltpu
from jax.experimental.pallas import tpu_sc as plsc

N = 10000
D_IN = 128
H = 64
E = 320000

NC = 2
NS = 16
NW = NC * NS

LCHUNK = 128
CPW = 80
EP = NW * LCHUNK * CPW
C0PW = 152
C1PW = 8
DUMMY = N

NP = 10240
RPS = NP // NS
BLK = 512
NBLK = NP // BLK
DEGW = 16

_MESH = dict(core_axis_name="c", subcore_axis_name="s",
             num_cores=NC, num_subcores=NS)

_SC_PARAMS = pltpu.CompilerParams(use_tc_tiling_on_sc=False)


def _sc_degree(dst2d, ones_d, zeros_d):
    @functools.partial(
        pl.kernel,
        out_type=jax.ShapeDtypeStruct((NC, NP, DEGW), jnp.float32),
        mesh=plsc.VectorSubcoreMesh(**_MESH),
        compiler_params=_SC_PARAMS,
        scratch_types=[
            pltpu.VMEM((CPW, LCHUNK), jnp.int32),
            pltpu.VMEM((LCHUNK, DEGW), jnp.float32),
            pltpu.VMEM_SHARED((NP, DEGW), jnp.float32),
        ],
    )
    def k(dst_hbm, ones_hbm, zeros_hbm, out_hbm, dst_v, ones_v, acc):
        cid = lax.axis_index("c")
        sid = lax.axis_index("s")
        wid = sid * NC + cid
        r0 = sid * RPS
        pltpu.sync_copy(zeros_hbm.at[pl.ds(r0, RPS)], acc.at[pl.ds(r0, RPS)])
        pltpu.sync_copy(ones_hbm, ones_v)
        pltpu.sync_copy(dst_hbm.at[pl.ds(wid * CPW, CPW)], dst_v)
        plsc.subcore_barrier()

        @pl.loop(0, CPW)
        def _(j):
            pltpu.sync_copy(ones_v, acc.at[dst_v.at[j]], add=True)

        plsc.subcore_barrier()
        pltpu.sync_copy(acc.at[pl.ds(r0, RPS)],
                        out_hbm.at[cid].at[pl.ds(r0, RPS)])

    return k(dst2d, ones_d, zeros_d)


def _sc_scatter(table, src2d, dst2d, zeros_hh):
    HH = H // 2

    @functools.partial(
        pl.kernel,
        out_type=jax.ShapeDtypeStruct((NC, 2, NP, HH), jnp.float32),
        mesh=plsc.VectorSubcoreMesh(**_MESH),
        compiler_params=_SC_PARAMS,
        scratch_types=[
            pltpu.VMEM((CPW, LCHUNK), jnp.int32),
            pltpu.VMEM((CPW, LCHUNK), jnp.int32),
            [pltpu.VMEM((LCHUNK, HH), jnp.float32) for _ in range(4)],
            pltpu.VMEM_SHARED((NP, HH), jnp.float32),
            pltpu.VMEM_SHARED((NP, HH), jnp.float32),
            [pltpu.SemaphoreType.DMA for _ in range(4)],
        ],
    )
    def k(table_hbm, src_hbm, dst_hbm, zeros_hbm, out_hbm,
          src_v, dst_v, rows_bufs, acc, tbl, sems):
        cid = lax.axis_index("c")
        sid = lax.axis_index("s")
        r0 = sid * RPS
        base = (cid * NS + sid) * CPW
        pltpu.sync_copy(src_hbm.at[pl.ds(base, CPW)], src_v)
        pltpu.sync_copy(dst_hbm.at[pl.ds(base, CPW)], dst_v)

        for half in range(2):
            pltpu.sync_copy(zeros_hbm.at[pl.ds(r0, RPS)],
                            acc.at[pl.ds(r0, RPS)])
            pltpu.sync_copy(
                table_hbm.at[pl.ds(r0, RPS), pl.ds(half * HH, HH)],
                tbl.at[pl.ds(r0, RPS)])
            plsc.subcore_barrier()

            for b in range(4):
                pltpu.async_copy(tbl.at[src_v.at[b]], rows_bufs[b], sems[b])

            @pl.loop(0, CPW, step=4)
            def _(j):
                for b in range(4):
                    pltpu.make_async_copy(tbl.at[src_v.at[0]],
                                          rows_bufs[b], sems[b]).wait()
                    pltpu.sync_copy(rows_bufs[b], acc.at[dst_v.at[j + b]],
                                    add=True)

                    @pl.when(j + b + 4 < CPW)
                    def _():
                        pltpu.async_copy(tbl.at[src_v.at[j + b + 4]],
                                         rows_bufs[b], sems[b])

            plsc.subcore_barrier()
            pltpu.sync_copy(acc.at[pl.ds(r0, RPS)],
                            out_hbm.at[cid].at[half].at[pl.ds(r0, RPS)])
            plsc.subcore_barrier()

    return k(table, src2d, dst2d, zeros_hh)


def _dinv_block(d_ref):
    s = jnp.sum(d_ref[0] + d_ref[1], axis=1, keepdims=True)
    return lax.rsqrt(s * (1.0 / DEGW) + 1.0)


_PREC = None


def _tc_prep(x_pad, W1, degp):
    def body(x_ref, w_ref, d_ref, g_ref):
        dinv = _dinv_block(d_ref)
        g_ref[...] = jnp.dot(x_ref[...], w_ref[...],
                             preferred_element_type=jnp.float32,
                             precision=_PREC) * dinv

    return pl.pallas_call(
        body,
        grid=(NBLK,),
        in_specs=[
            pl.BlockSpec((BLK, D_IN), lambda i: (i, 0)),
            pl.BlockSpec((D_IN, H), lambda i: (0, 0)),
            pl.BlockSpec((NC, BLK, DEGW), lambda i: (0, i, 0)),
        ],
        out_specs=pl.BlockSpec((BLK, H), lambda i: (i, 0)),
        out_shape=jax.ShapeDtypeStruct((NP, H), jnp.float32),
    )(x_pad, W1, degp)


def _s_block(s_ref):
    lo = s_ref[0, 0] + s_ref[1, 0]
    hi = s_ref[0, 1] + s_ref[1, 1]
    return jnp.concatenate([lo, hi], axis=1)


def _pre_block(s_ref, g_ref, d_ref, b_ref):
    dinv = _dinv_block(d_ref)
    return dinv * (_s_block(s_ref) + g_ref[...]) + b_ref[...]


def _tc_stats(Sp, g, degp, b2d):
    def body(s_ref, g_ref, d_ref, b_ref, o_ref, acc_ref):
        i = pl.program_id(0)

        @pl.when(i == 0)
        def _():
            acc_ref[0] = 0.0
            acc_ref[1] = 0.0

        pre = _pre_block(s_ref, g_ref, d_ref, b_ref)
        rows = lax.broadcasted_iota(jnp.int32, (BLK, H), 0) + i * BLK
        prem = jnp.where(rows < N, pre, 0.0)
        acc_ref[0] += jnp.sum(prem)
        acc_ref[1] += jnp.sum(prem * prem)

        @pl.when(i == NBLK - 1)
        def _():
            cnt = float(N * H)
            mean = acc_ref[0] / cnt
            var = acc_ref[1] / cnt - mean * mean
            o_ref[0] = mean
            o_ref[1] = lax.rsqrt(var + 1e-5)

    return pl.pallas_call(
        body,
        grid=(NBLK,),
        in_specs=[
            pl.BlockSpec((NC, 2, BLK, H // 2), lambda i: (0, 0, i, 0)),
            pl.BlockSpec((BLK, H), lambda i: (i, 0)),
            pl.BlockSpec((NC, BLK, DEGW), lambda i: (0, i, 0)),
            pl.BlockSpec((1, H), lambda i: (0, 0)),
        ],
        out_specs=pl.BlockSpec(memory_space=pltpu.SMEM),
        out_shape=jax.ShapeDtypeStruct((2,), jnp.float32),
        scratch_shapes=[pltpu.SMEM((2,), jnp.float32)],
    )(Sp, g, degp, b2d)


def _tc_epilogue(Sp, g, degp, b2d, st, lnw2d, lnb2d, h_prev, W_next):
    residual = h_prev is not None

    def body(s_ref, g_ref, d_ref, b_ref, lnw_ref, lnb_ref, *rest):
        if residual:
            hp_ref, w_ref, st_ref, h_ref, gn_ref = rest
        else:
            w_ref, st_ref, h_ref, gn_ref = rest
        dinv = _dinv_block(d_ref)
        pre = dinv * (_s_block(s_ref) + g_ref[...]) + b_ref[...]
        y = (pre - st_ref[0]) * st_ref[1] * lnw_ref[...] + lnb_ref[...]
        h = jnp.maximum(y, 0.0)
        if residual:
            h = h + hp_ref[...]
        h_ref[...] = h
        gn_ref[...] = jnp.dot(h, w_ref[...],
                              preferred_element_type=jnp.float32,
                              precision=_PREC) * dinv

    in_specs = [
        pl.BlockSpec((NC, 2, BLK, H // 2), lambda i: (0, 0, i, 0)),
        pl.BlockSpec((BLK, H), lambda i: (i, 0)),
        pl.BlockSpec((NC, BLK, DEGW), lambda i: (0, i, 0)),
        pl.BlockSpec((1, H), lambda i: (0, 0)),
        pl.BlockSpec((1, H), lambda i: (0, 0)),
        pl.BlockSpec((1, H), lambda i: (0, 0)),
    ]
    args = [Sp, g, degp, b2d, lnw2d, lnb2d]
    if residual:
        in_specs.append(pl.BlockSpec((BLK, H), lambda i: (i, 0)))
        args.append(h_prev)
    in_specs.append(pl.BlockSpec((H, H), lambda i: (0, 0)))
    args.append(W_next)
    in_specs.append(pl.BlockSpec(memory_space=pltpu.SMEM))
    args.append(st)

    return pl.pallas_call(
        body,
        grid=(NBLK,),
        in_specs=in_specs,
        out_specs=[
            pl.BlockSpec((BLK, H), lambda i: (i, 0)),
            pl.BlockSpec((BLK, H), lambda i: (i, 0)),
        ],
        out_shape=[
            jax.ShapeDtypeStruct((NP, H), jnp.float32),
            jax.ShapeDtypeStruct((NP, H), jnp.float32),
        ],
    )(*args)


def _tc_epilogue_head(Sp, g, degp, b2d, st, lnw2d, lnb2d, h_prev,
                      Wp1, bp1_2d, Wp2_row, bp2):
    def body(s_ref, g_ref, d_ref, b_ref, lnw_ref, lnb_ref, hp_ref,
             wp1_ref, bp1_ref, wp2_ref, st_ref, bp2_ref,
             emb_ref, pk_ref):
        pre = _pre_block(s_ref, g_ref, d_ref, b_ref)
        y = (pre - st_ref[0]) * st_ref[1] * lnw_ref[...] + lnb_ref[...]
        h = jnp.maximum(y, 0.0) + hp_ref[...]
        emb_ref[...] = h
        t = jnp.maximum(
            jnp.dot(h, wp1_ref[...], preferred_element_type=jnp.float32,
                    precision=_PREC) + bp1_ref[...], 0.0)
        pk_ref[...] = jnp.sum(t * wp2_ref[...], axis=1) + bp2_ref[0]

    return pl.pallas_call(
        body,
        grid=(NBLK,),
        in_specs=[
            pl.BlockSpec((NC, 2, BLK, H // 2), lambda i: (0, 0, i, 0)),
            pl.BlockSpec((BLK, H), lambda i: (i, 0)),
            pl.BlockSpec((NC, BLK, DEGW), lambda i: (0, i, 0)),
            pl.BlockSpec((1, H), lambda i: (0, 0)),
            pl.BlockSpec((1, H), lambda i: (0, 0)),
            pl.BlockSpec((1, H), lambda i: (0, 0)),
            pl.BlockSpec((BLK, H), lambda i: (i, 0)),
            pl.BlockSpec((H, H // 2), lambda i: (0, 0)),
            pl.BlockSpec((1, H // 2), lambda i: (0, 0)),
            pl.BlockSpec((1, H // 2), lambda i: (0, 0)),
            pl.BlockSpec(memory_space=pltpu.SMEM),
            pl.BlockSpec(memory_space=pltpu.SMEM),
        ],
        out_specs=[
            pl.BlockSpec((BLK, H), lambda i: (i, 0)),
            pl.BlockSpec((BLK,), lambda i: (i,)),
        ],
        out_shape=[
            jax.ShapeDtypeStruct((NP, H), jnp.float32),
            jax.ShapeDtypeStruct((NP,), jnp.float32),
        ],
    )(Sp, g, degp, b2d, lnw2d, lnb2d, h_prev, Wp1, bp1_2d, Wp2_row, st, bp2)


def kernel(x, edge_index, W1, b1, ln_w1, ln_b1, W2, b2, ln_w2, ln_b2,
           W3, b3, ln_w3, ln_b3, Wp1, bp1, Wp2, bp2):
    src = edge_index[0]
    dst = edge_index[1]
    pad = jnp.full((EP - E,), DUMMY, dtype=src.dtype)
    src2d = jnp.concatenate([src, pad]).reshape(NW * CPW, LCHUNK)
    dst2d = jnp.concatenate([dst, pad]).reshape(NW * CPW, LCHUNK)

    x_pad = jnp.pad(x, ((0, NP - N), (0, 0)))
    zeros_h = jnp.zeros((NP, H // 2), jnp.float32)
    zeros_d = jnp.zeros((NP, DEGW), jnp.float32)
    ones_d = jnp.ones((LCHUNK, DEGW), jnp.float32)

    b1r = b1.reshape(1, H)
    b2r = b2.reshape(1, H)
    b3r = b3.reshape(1, H)
    lnw1r = ln_w1.reshape(1, H)
    lnb1r = ln_b1.reshape(1, H)
    lnw2r = ln_w2.reshape(1, H)
    lnb2r = ln_b2.reshape(1, H)
    lnw3r = ln_w3.reshape(1, H)
    lnb3r = ln_b3.reshape(1, H)
    bp1r = bp1.reshape(1, H // 2)
    wp2r = Wp2.reshape(1, H // 2)

    degp = _sc_degree(dst2d, ones_d, zeros_d)

    g1 = _tc_prep(x_pad, W1, degp)
    S1 = _sc_scatter(g1, src2d, dst2d, zeros_h)
    st1 = _tc_stats(S1, g1, degp, b1r)
    h1, g2 = _tc_epilogue(S1, g1, degp, b1r, st1, lnw1r, lnb1r, None, W2)

    S2 = _sc_scatter(g2, src2d, dst2d, zeros_h)
    st2 = _tc_stats(S2, g2, degp, b2r)
    h2, g3 = _tc_epilogue(S2, g2, degp, b2r, st2, lnw2r, lnb2r, h1, W3)

    S3 = _sc_scatter(g3, src2d, dst2d, zeros_h)
    st3 = _tc_stats(S3, g3, degp, b3r)
    emb, pk = _tc_epilogue_head(S3, g3, degp, b3r, st3, lnw3r, lnb3r, h2,
                                Wp1, bp1r, wp2r, bp2)

    return (emb[:N], pk[:N, None])


# speedup vs baseline: 2.0431x; 1.0004x over previous
import functools

import jax
import jax.numpy as jnp
from jax import lax
from jax.experimental import pallas as pl
from jax.experimental.pallas import tpu as pltpu
from jax.experimental.pallas import tpu_sc as plsc

N = 10000
D_IN = 128
H = 64
E = 320000

NC = 2
NS = 16
NW = NC * NS

LCHUNK = 128
CPW = 80
EP = NW * LCHUNK * CPW
DUMMY = N

NP = 10240
RPS = NP // NS
BLK = 512
NBLK = NP // BLK
DEGW = 16

_MESH = dict(core_axis_name="c", subcore_axis_name="s",
             num_cores=NC, num_subcores=NS)

_SC_PARAMS = pltpu.CompilerParams(use_tc_tiling_on_sc=False)


def _sc_degree(dst2d, ones_d, zeros_d):
    @functools.partial(
        pl.kernel,
        out_type=jax.ShapeDtypeStruct((NC, NP, DEGW), jnp.float32),
        mesh=plsc.VectorSubcoreMesh(**_MESH),
        compiler_params=_SC_PARAMS,
        scratch_types=[
            pltpu.VMEM((CPW, LCHUNK), jnp.int32),
            pltpu.VMEM((LCHUNK, DEGW), jnp.float32),
            pltpu.VMEM_SHARED((NP, DEGW), jnp.float32),
        ],
    )
    def k(dst_hbm, ones_hbm, zeros_hbm, out_hbm, dst_v, ones_v, acc):
        cid = lax.axis_index("c")
        sid = lax.axis_index("s")
        wid = sid * NC + cid
        r0 = sid * RPS
        pltpu.sync_copy(zeros_hbm.at[pl.ds(r0, RPS)], acc.at[pl.ds(r0, RPS)])
        pltpu.sync_copy(ones_hbm, ones_v)
        pltpu.sync_copy(dst_hbm.at[pl.ds(wid * CPW, CPW)], dst_v)
        plsc.subcore_barrier()

        @pl.loop(0, CPW)
        def _(j):
            pltpu.sync_copy(ones_v, acc.at[dst_v.at[j]], add=True)

        plsc.subcore_barrier()
        pltpu.sync_copy(acc.at[pl.ds(r0, RPS)],
                        out_hbm.at[cid].at[pl.ds(r0, RPS)])

    return k(dst2d, ones_d, zeros_d)


def _sc_scatter(table, src2d, dst2d, zeros_hh):
    HH = H // 2

    @functools.partial(
        pl.kernel,
        out_type=jax.ShapeDtypeStruct((NC, 2, NP, HH), jnp.float32),
        mesh=plsc.VectorSubcoreMesh(**_MESH),
        compiler_params=_SC_PARAMS,
        scratch_types=[
            pltpu.VMEM((CPW, LCHUNK), jnp.int32),
            pltpu.VMEM((CPW, LCHUNK), jnp.int32),
            [pltpu.VMEM((LCHUNK, HH), jnp.float32) for _ in range(4)],
            pltpu.VMEM_SHARED((NP, HH), jnp.float32),
            pltpu.VMEM_SHARED((NP, HH), jnp.float32),
            [pltpu.SemaphoreType.DMA for _ in range(4)],
        ],
    )
    def k(table_hbm, src_hbm, dst_hbm, zeros_hbm, out_hbm,
          src_v, dst_v, rows_bufs, acc, tbl, sems):
        cid = lax.axis_index("c")
        sid = lax.axis_index("s")
        r0 = sid * RPS
        base = (cid * NS + sid) * CPW
        pltpu.sync_copy(src_hbm.at[pl.ds(base, CPW)], src_v)
        pltpu.sync_copy(dst_hbm.at[pl.ds(base, CPW)], dst_v)

        for half in range(2):
            pltpu.sync_copy(zeros_hbm.at[pl.ds(r0, RPS)],
                            acc.at[pl.ds(r0, RPS)])
            pltpu.sync_copy(
                table_hbm.at[pl.ds(r0, RPS), pl.ds(half * HH, HH)],
                tbl.at[pl.ds(r0, RPS)])
            plsc.subcore_barrier()

            for b in range(4):
                pltpu.async_copy(tbl.at[src_v.at[b]], rows_bufs[b], sems[b])

            @pl.loop(0, CPW, step=4)
            def _(j):
                for b in range(4):
                    pltpu.make_async_copy(tbl.at[src_v.at[0]],
                                          rows_bufs[b], sems[b]).wait()
                    pltpu.sync_copy(rows_bufs[b], acc.at[dst_v.at[j + b]],
                                    add=True)

                    @pl.when(j + b + 4 < CPW)
                    def _():
                        pltpu.async_copy(tbl.at[src_v.at[j + b + 4]],
                                         rows_bufs[b], sems[b])

            plsc.subcore_barrier()
            pltpu.sync_copy(acc.at[pl.ds(r0, RPS)],
                            out_hbm.at[cid].at[half].at[pl.ds(r0, RPS)])
            plsc.subcore_barrier()

    return k(table, src2d, dst2d, zeros_hh)


def _dinv_block(d_ref):
    s = jnp.sum(d_ref[0] + d_ref[1], axis=1, keepdims=True)
    return lax.rsqrt(s * (1.0 / DEGW) + 1.0)


_PREC = None


def _tc_prep(x_pad, W1, degp):
    def body(x_ref, w_ref, d_ref, g_ref):
        dinv = _dinv_block(d_ref)
        g_ref[...] = jnp.dot(x_ref[...], w_ref[...],
                             preferred_element_type=jnp.float32,
                             precision=_PREC) * dinv

    return pl.pallas_call(
        body,
        grid=(NBLK,),
        in_specs=[
            pl.BlockSpec((BLK, D_IN), lambda i: (i, 0)),
            pl.BlockSpec((D_IN, H), lambda i: (0, 0)),
            pl.BlockSpec((NC, BLK, DEGW), lambda i: (0, i, 0)),
        ],
        out_specs=pl.BlockSpec((BLK, H), lambda i: (i, 0)),
        out_shape=jax.ShapeDtypeStruct((NP, H), jnp.float32),
    )(x_pad, W1, degp)


def _s_block(s_ref):
    lo = s_ref[0, 0] + s_ref[1, 0]
    hi = s_ref[0, 1] + s_ref[1, 1]
    return jnp.concatenate([lo, hi], axis=1)


def _pre_block(s_ref, g_ref, d_ref, b_ref):
    dinv = _dinv_block(d_ref)
    return dinv * (_s_block(s_ref) + g_ref[...]) + b_ref[...]


def _tc_stats(Sp, g, degp, b2d):
    def body(s_ref, g_ref, d_ref, b_ref, o_ref, acc_ref):
        i = pl.program_id(0)

        @pl.when(i == 0)
        def _():
            acc_ref[0] = 0.0
            acc_ref[1] = 0.0

        pre = _pre_block(s_ref, g_ref, d_ref, b_ref)
        rows = lax.broadcasted_iota(jnp.int32, (BLK, H), 0) + i * BLK
        prem = jnp.where(rows < N, pre, 0.0)
        acc_ref[0] += jnp.sum(prem)
        acc_ref[1] += jnp.sum(prem * prem)

        @pl.when(i == NBLK - 1)
        def _():
            cnt = float(N * H)
            mean = acc_ref[0] / cnt
            var = acc_ref[1] / cnt - mean * mean
            o_ref[0] = mean
            o_ref[1] = lax.rsqrt(var + 1e-5)

    return pl.pallas_call(
        body,
        grid=(NBLK,),
        in_specs=[
            pl.BlockSpec((NC, 2, BLK, H // 2), lambda i: (0, 0, i, 0)),
            pl.BlockSpec((BLK, H), lambda i: (i, 0)),
            pl.BlockSpec((NC, BLK, DEGW), lambda i: (0, i, 0)),
            pl.BlockSpec((1, H), lambda i: (0, 0)),
        ],
        out_specs=pl.BlockSpec(memory_space=pltpu.SMEM),
        out_shape=jax.ShapeDtypeStruct((2,), jnp.float32),
        scratch_shapes=[pltpu.SMEM((2,), jnp.float32)],
    )(Sp, g, degp, b2d)


def _tc_epilogue(Sp, g, degp, b2d, st, lnw2d, lnb2d, h_prev, W_next):
    residual = h_prev is not None

    def body(s_ref, g_ref, d_ref, b_ref, lnw_ref, lnb_ref, *rest):
        if residual:
            hp_ref, w_ref, st_ref, h_ref, gn_ref = rest
        else:
            w_ref, st_ref, h_ref, gn_ref = rest
        dinv = _dinv_block(d_ref)
        pre = dinv * (_s_block(s_ref) + g_ref[...]) + b_ref[...]
        y = (pre - st_ref[0]) * st_ref[1] * lnw_ref[...] + lnb_ref[...]
        h = jnp.maximum(y, 0.0)
        if residual:
            h = h + hp_ref[...]
        h_ref[...] = h
        gn_ref[...] = jnp.dot(h, w_ref[...],
                              preferred_element_type=jnp.float32,
                              precision=_PREC) * dinv

    in_specs = [
        pl.BlockSpec((NC, 2, BLK, H // 2), lambda i: (0, 0, i, 0)),
        pl.BlockSpec((BLK, H), lambda i: (i, 0)),
        pl.BlockSpec((NC, BLK, DEGW), lambda i: (0, i, 0)),
        pl.BlockSpec((1, H), lambda i: (0, 0)),
        pl.BlockSpec((1, H), lambda i: (0, 0)),
        pl.BlockSpec((1, H), lambda i: (0, 0)),
    ]
    args = [Sp, g, degp, b2d, lnw2d, lnb2d]
    if residual:
        in_specs.append(pl.BlockSpec((BLK, H), lambda i: (i, 0)))
        args.append(h_prev)
    in_specs.append(pl.BlockSpec((H, H), lambda i: (0, 0)))
    args.append(W_next)
    in_specs.append(pl.BlockSpec(memory_space=pltpu.SMEM))
    args.append(st)

    return pl.pallas_call(
        body,
        grid=(NBLK,),
        in_specs=in_specs,
        out_specs=[
            pl.BlockSpec((BLK, H), lambda i: (i, 0)),
            pl.BlockSpec((BLK, H), lambda i: (i, 0)),
        ],
        out_shape=[
            jax.ShapeDtypeStruct((NP, H), jnp.float32),
            jax.ShapeDtypeStruct((NP, H), jnp.float32),
        ],
    )(*args)


def _tc_epilogue_head(Sp, g, degp, b2d, st, lnw2d, lnb2d, h_prev,
                      Wp1, bp1_2d, Wp2_row, bp2):
    def body(s_ref, g_ref, d_ref, b_ref, lnw_ref, lnb_ref, hp_ref,
             wp1_ref, bp1_ref, wp2_ref, st_ref, bp2_ref,
             emb_ref, pk_ref):
        pre = _pre_block(s_ref, g_ref, d_ref, b_ref)
        y = (pre - st_ref[0]) * st_ref[1] * lnw_ref[...] + lnb_ref[...]
        h = jnp.maximum(y, 0.0) + hp_ref[...]
        emb_ref[...] = h
        t = jnp.maximum(
            jnp.dot(h, wp1_ref[...], preferred_element_type=jnp.float32,
                    precision=_PREC) + bp1_ref[...], 0.0)
        pk_ref[...] = jnp.sum(t * wp2_ref[...], axis=1) + bp2_ref[0]

    return pl.pallas_call(
        body,
        grid=(NBLK,),
        in_specs=[
            pl.BlockSpec((NC, 2, BLK, H // 2), lambda i: (0, 0, i, 0)),
            pl.BlockSpec((BLK, H), lambda i: (i, 0)),
            pl.BlockSpec((NC, BLK, DEGW), lambda i: (0, i, 0)),
            pl.BlockSpec((1, H), lambda i: (0, 0)),
            pl.BlockSpec((1, H), lambda i: (0, 0)),
            pl.BlockSpec((1, H), lambda i: (0, 0)),
            pl.BlockSpec((BLK, H), lambda i: (i, 0)),
            pl.BlockSpec((H, H // 2), lambda i: (0, 0)),
            pl.BlockSpec((1, H // 2), lambda i: (0, 0)),
            pl.BlockSpec((1, H // 2), lambda i: (0, 0)),
            pl.BlockSpec(memory_space=pltpu.SMEM),
            pl.BlockSpec(memory_space=pltpu.SMEM),
        ],
        out_specs=[
            pl.BlockSpec((BLK, H), lambda i: (i, 0)),
            pl.BlockSpec((BLK,), lambda i: (i,)),
        ],
        out_shape=[
            jax.ShapeDtypeStruct((NP, H), jnp.float32),
            jax.ShapeDtypeStruct((NP,), jnp.float32),
        ],
    )(Sp, g, degp, b2d, lnw2d, lnb2d, h_prev, Wp1, bp1_2d, Wp2_row, st, bp2)


def kernel(x, edge_index, W1, b1, ln_w1, ln_b1, W2, b2, ln_w2, ln_b2,
           W3, b3, ln_w3, ln_b3, Wp1, bp1, Wp2, bp2):
    src = edge_index[0]
    dst = edge_index[1]
    pad = jnp.full((EP - E,), DUMMY, dtype=src.dtype)
    src2d = jnp.concatenate([src, pad]).reshape(NW * CPW, LCHUNK)
    dst2d = jnp.concatenate([dst, pad]).reshape(NW * CPW, LCHUNK)

    x_pad = jnp.pad(x, ((0, NP - N), (0, 0)))
    zeros_h = jnp.zeros((NP, H // 2), jnp.float32)
    zeros_d = jnp.zeros((NP, DEGW), jnp.float32)
    ones_d = jnp.ones((LCHUNK, DEGW), jnp.float32)

    b1r = b1.reshape(1, H)
    b2r = b2.reshape(1, H)
    b3r = b3.reshape(1, H)
    lnw1r = ln_w1.reshape(1, H)
    lnb1r = ln_b1.reshape(1, H)
    lnw2r = ln_w2.reshape(1, H)
    lnb2r = ln_b2.reshape(1, H)
    lnw3r = ln_w3.reshape(1, H)
    lnb3r = ln_b3.reshape(1, H)
    bp1r = bp1.reshape(1, H // 2)
    wp2r = Wp2.reshape(1, H // 2)

    degp = _sc_degree(dst2d, ones_d, zeros_d)

    g1 = _tc_prep(x_pad, W1, degp)
    S1 = _sc_scatter(g1, src2d, dst2d, zeros_h)
    st1 = _tc_stats(S1, g1, degp, b1r)
    h1, g2 = _tc_epilogue(S1, g1, degp, b1r, st1, lnw1r, lnb1r, None, W2)

    S2 = _sc_scatter(g2, src2d, dst2d, zeros_h)
    st2 = _tc_stats(S2, g2, degp, b2r)
    h2, g3 = _tc_epilogue(S2, g2, degp, b2r, st2, lnw2r, lnb2r, h1, W3)

    S3 = _sc_scatter(g3, src2d, dst2d, zeros_h)
    st3 = _tc_stats(S3, g3, degp, b3r)
    emb, pk = _tc_epilogue_head(S3, g3, degp, b3r, st3, lnw3r, lnb3r, h2,
                                Wp1, bp1r, wp2r, bp2)

    return (emb[:N], pk[:N, None])
